# baseline (speedup 1.0000x reference)
import functools

import jax
import jax.numpy as jnp
from jax import lax
from jax.experimental import pallas as pl
from jax.experimental.pallas import tpu as pltpu
from jax.experimental.pallas import tpu_sc as plsc

_NC = 2
_NS = 16
_L = 16
_NW = _NC * _NS
_CHUNK = 16384
_NBUF = 2
_UNROLL = 4


@functools.lru_cache(maxsize=None)
def _build(total_elems: int):
    assert total_elems % (_NW * _CHUNK * _NBUF) == 0
    epw = total_elems // _NW
    nchunks = epw // _CHUNK
    vecs = _CHUNK // _L
    mesh = plsc.VectorSubcoreMesh(core_axis_name="c", subcore_axis_name="s")

    @functools.partial(
        pl.kernel,
        out_type=jax.ShapeDtypeStruct((_NW, _L), jnp.float32),
        mesh=mesh,
        scratch_types=[
            pltpu.VMEM((_NBUF, _CHUNK), jnp.float32),
            pltpu.VMEM((_NBUF, _CHUNK), jnp.float32),
            pltpu.VMEM((_L,), jnp.float32),
            pltpu.SemaphoreType.DMA,
            pltpu.SemaphoreType.DMA,
            pltpu.SemaphoreType.DMA,
            pltpu.SemaphoreType.DMA,
        ],
    )
    def masked_mse_partials(o_hbm, t_hbm, out_hbm, obuf, tbuf, accv,
                            so0, so1, st0, st1):
        osems = (so0, so1)
        tsems = (st0, st1)
        wid = lax.axis_index("s") * _NC + lax.axis_index("c")
        base = wid * epw

        def start(ci, b):
            off = pl.multiple_of(base + ci * _CHUNK, _CHUNK)
            pltpu.async_copy(o_hbm.at[pl.ds(off, _CHUNK)], obuf.at[b], osems[b])
            pltpu.async_copy(t_hbm.at[pl.ds(off, _CHUNK)], tbuf.at[b], tsems[b])

        def wait(b):
            pltpu.make_async_copy(
                o_hbm.at[pl.ds(0, _CHUNK)], obuf.at[b], osems[b]).wait()
            pltpu.make_async_copy(
                t_hbm.at[pl.ds(0, _CHUNK)], tbuf.at[b], tsems[b]).wait()

        def consume(b, accs):
            def body(j, accs):
                jb = pl.multiple_of(j * (_UNROLL * _L), _L)
                new = []
                for u in range(_UNROLL):
                    idx = pl.ds(jb + u * _L, _L)
                    o = obuf[b, idx]
                    t = tbuf[b, idx]
                    d = o - t
                    sq = d * d
                    new.append(accs[u] + jnp.where(t != 0.0, sq, 0.0))
                return tuple(new)
            return lax.fori_loop(0, vecs // _UNROLL, body, accs)

        for b in range(_NBUF):
            start(b, b)

        zeros = jnp.zeros((_L,), jnp.float32)
        accs0 = (zeros,) * _UNROLL

        def outer(i, accs):
            for b in range(_NBUF):
                ci = i * _NBUF + b
                wait(b)
                accs = consume(b, accs)

                @pl.when(ci + _NBUF < nchunks)
                def _():
                    start(ci + _NBUF, b)
            return accs

        accs = lax.fori_loop(0, nchunks // _NBUF, outer, accs0)
        total = accs[0] + accs[1] + accs[2] + accs[3]
        accv[...] = total
        pltpu.sync_copy(accv, out_hbm.at[wid])

    return masked_mse_partials


def kernel(output, target):
    total = output.size
    o = output.reshape(total)
    t = target.reshape(total)
    partials = _build(total)(o, t)
    return jnp.sum(partials) / jnp.float32(total)
